# Initial kernel scaffold; baseline (speedup 1.0000x reference)
import jax
import jax.numpy as jnp
from jax.experimental import pallas as pl


def kernel(x, W1, S1, att_src1, att_dst1, W2, graph_edges):
    raise NotImplementedError("write your pallas kernel here")


# speedup vs baseline: 13.3759x; 13.3759x over previous
import functools

import jax
import jax.numpy as jnp
from jax import lax
from jax.experimental import pallas as pl
from jax.experimental.pallas import tpu as pltpu
from jax.experimental.pallas import tpu_sc as plsc

N = 10000
E = 320000
D_IN, D_LAT, D_EMB = 128, 64, 32

NCORE, NSUB = 2, 16
CH = 80
NCH = 125
EPT = CH * NCH
ROWB = 125
RSTRIPE = N // NSUB
TBLK = 1000


def _enc_body(x_ref, w1_ref, s1_ref, avs_ref, avd_ref, hs_ref, as_ref, ad_ref):
    xb = x_ref[...]
    hs = jnp.dot(xb, w1_ref[...], preferred_element_type=jnp.float32)
    hd = jnp.dot(xb, s1_ref[...], preferred_element_type=jnp.float32)
    hs_ref[...] = hs
    as_ref[...] = jnp.sum(hs * avs_ref[...][None, :], axis=1)[:, None]
    ad_ref[...] = jnp.sum(hd * avd_ref[...][None, :], axis=1)[:, None]


def _enc_stage(x, W1, S1, avs, avd):
    grid = (N // TBLK,)
    return pl.pallas_call(
        _enc_body,
        grid=grid,
        in_specs=[
            pl.BlockSpec((TBLK, D_IN), lambda i: (i, 0)),
            pl.BlockSpec((D_IN, D_LAT), lambda i: (0, 0)),
            pl.BlockSpec((D_IN, D_LAT), lambda i: (0, 0)),
            pl.BlockSpec((D_LAT,), lambda i: (0,)),
            pl.BlockSpec((D_LAT,), lambda i: (0,)),
        ],
        out_specs=[
            pl.BlockSpec((TBLK, D_LAT), lambda i: (i, 0)),
            pl.BlockSpec((TBLK, 1), lambda i: (i, 0)),
            pl.BlockSpec((TBLK, 1), lambda i: (i, 0)),
        ],
        out_shape=[
            jax.ShapeDtypeStruct((N, D_LAT), jnp.float32),
            jax.ShapeDtypeStruct((N, 1), jnp.float32),
            jax.ShapeDtypeStruct((N, 1), jnp.float32),
        ],
    )(x, W1, S1, avs, avd)


def _mid_body(p0_ref, p1_ref, s_ref, w2_ref, emb_ref, hd_ref):
    t = (p0_ref[...] + p1_ref[...]) / (s_ref[...] + 1e-16)
    h1 = jnp.where(t > 0, t, jnp.exp(t) - 1.0)
    w2 = w2_ref[...]
    emb = jnp.dot(h1, w2, preferred_element_type=jnp.float32)
    emb_ref[...] = emb
    hd_ref[...] = lax.dot_general(emb, w2, (((1,), (1,)), ((), ())),
                                  preferred_element_type=jnp.float32)


def _mid_stage(p0, p1, s, W2):
    grid = (N // TBLK,)
    return pl.pallas_call(
        _mid_body,
        grid=grid,
        in_specs=[
            pl.BlockSpec((TBLK, D_LAT), lambda i: (i, 0)),
            pl.BlockSpec((TBLK, D_LAT), lambda i: (i, 0)),
            pl.BlockSpec((TBLK, 1), lambda i: (i, 0)),
            pl.BlockSpec((D_LAT, D_EMB), lambda i: (0, 0)),
        ],
        out_specs=[
            pl.BlockSpec((TBLK, D_EMB), lambda i: (i, 0)),
            pl.BlockSpec((TBLK, D_LAT), lambda i: (i, 0)),
        ],
        out_shape=[
            jax.ShapeDtypeStruct((N, D_EMB), jnp.float32),
            jax.ShapeDtypeStruct((N, D_LAT), jnp.float32),
        ],
    )(p0, p1, s, W2)


def _dec_body(p0_ref, p1_ref, s_ref, w1_ref, rec_ref):
    t = (p0_ref[...] + p1_ref[...]) / (s_ref[...] + 1e-16)
    d1 = jnp.where(t > 0, t, jnp.exp(t) - 1.0)
    rec_ref[...] = lax.dot_general(d1, w1_ref[...], (((1,), (1,)), ((), ())),
                                   preferred_element_type=jnp.float32)


def _dec_stage(p0, p1, s, W1):
    grid = (N // TBLK,)
    return pl.pallas_call(
        _dec_body,
        grid=grid,
        in_specs=[
            pl.BlockSpec((TBLK, D_LAT), lambda i: (i, 0)),
            pl.BlockSpec((TBLK, D_LAT), lambda i: (i, 0)),
            pl.BlockSpec((TBLK, 1), lambda i: (i, 0)),
            pl.BlockSpec((D_IN, D_LAT), lambda i: (0, 0)),
        ],
        out_specs=pl.BlockSpec((TBLK, D_IN), lambda i: (i, 0)),
        out_shape=jax.ShapeDtypeStruct((N, D_IN), jnp.float32),
    )(p0, p1, s, W1)


def _zero_vmem(ref, nrow):
    z = jnp.zeros((16,), jnp.float32)

    def body(r, _):
        for q in range(ref.shape[1] // 16):
            ref[r, pl.ds(q * 16, 16)] = z
        return 0

    lax.fori_loop(0, nrow, body, 0)


def _p2_chunks(base, src_hbm, dst_hbm, srcc_v, dstc_v, w1_v, rows_v,
               feat_hbm, acc_sh, sem):
    def chunk(ch, _):
        e0 = base + ch * CH
        pltpu.sync_copy(src_hbm.at[pl.ds(e0, CH)], srcc_v)
        pltpu.sync_copy(dst_hbm.at[pl.ds(e0, CH)], dstc_v)
        pltpu.async_copy(feat_hbm.at[srcc_v], rows_v, sem).wait()
        for g in range(CH // 16):
            wvec = w1_v[pl.ds(ch * CH + g * 16, 16)]
            for k in range(16):
                wv = jnp.full((16,), wvec[k])
                e = g * 16 + k
                for q in range(D_LAT // 16):
                    rows_v[e, pl.ds(q * 16, 16)] = (
                        rows_v[e, pl.ds(q * 16, 16)] * wv)
        pltpu.sync_copy(rows_v, acc_sh.at[dstc_v], add=True)
        return 0

    lax.fori_loop(0, NCH, chunk, 0)


def _sc1_body(src_hbm, dst_hbm, asrc_hbm, adst_hbm, hsrc_hbm,
              w_hbm, s_hbm, acc_hbm,
              asrc_v, adst_v, srcc_v, dstc_v, w1_v, rows_v, wb_v,
              s_sh, acc_sh, sem):
    c = lax.axis_index("c")
    t = lax.axis_index("s")

    pltpu.sync_copy(asrc_hbm, asrc_v)
    pltpu.sync_copy(adst_hbm, adst_v)

    _zero_vmem(wb_v, RSTRIPE)
    pltpu.sync_copy(wb_v, acc_sh.at[pl.ds(t * RSTRIPE, RSTRIPE)])

    def zs(j, _):
        w1_v[pl.ds(j * 16, 16)] = jnp.zeros((16,), jnp.float32)
        return 0

    lax.fori_loop(0, 40, zs, 0)

    @pl.when(t < NSUB - 1)
    def _():
        pltpu.sync_copy(w1_v.at[pl.ds(0, 632)], s_sh.at[pl.ds(t * 632, 632)])

    @pl.when(t == NSUB - 1)
    def _():
        pltpu.sync_copy(w1_v.at[pl.ds(0, 520)], s_sh.at[pl.ds(15 * 632, 520)])

    plsc.subcore_barrier()

    def p1_pass(base):
        def chunk(ch, _):
            e0 = base + ch * CH
            pltpu.sync_copy(src_hbm.at[pl.ds(e0, CH)], srcc_v)
            pltpu.sync_copy(dst_hbm.at[pl.ds(e0, CH)], dstc_v)
            for j in range(CH // 16):
                s16 = srcc_v[pl.ds(j * 16, 16)]
                d16 = dstc_v[pl.ds(j * 16, 16)]
                a_s = plsc.load_gather(asrc_v, [s16])
                a_d = plsc.load_gather(adst_v, [d16])
                z = a_s + a_d
                sg = 1.0 / (1.0 + jnp.exp(-z))
                w1_v[pl.ds(ch * CH + j * 16, 16)] = jnp.exp(sg)
            pltpu.sync_copy(w1_v.at[pl.ds(ch * CH, CH)],
                            s_sh.at[dstc_v], add=True)
            return 0

        lax.fori_loop(0, NCH, chunk, 0)

    p1_pass(t * (2 * EPT) + (1 - c) * EPT)
    own_base = t * (2 * EPT) + c * EPT
    p1_pass(own_base)
    pltpu.sync_copy(w1_v, w_hbm.at[pl.ds(own_base, EPT)])

    _p2_chunks(own_base, src_hbm, dst_hbm, srcc_v, dstc_v, w1_v, rows_v,
               hsrc_hbm, acc_sh, sem)

    plsc.subcore_barrier()

    r0 = t * RSTRIPE
    pltpu.sync_copy(acc_sh.at[pl.ds(r0, RSTRIPE)], wb_v)
    pltpu.sync_copy(wb_v, acc_hbm.at[c, pl.ds(r0, RSTRIPE)])

    @pl.when(c == 0)
    def _():
        @pl.when(t < NSUB - 1)
        def _():
            pltpu.sync_copy(s_sh.at[pl.ds(t * 632, 632)],
                            w1_v.at[pl.ds(0, 632)])
            pltpu.sync_copy(w1_v.at[pl.ds(0, 632)],
                            s_hbm.at[pl.ds(t * 632, 632)])

        @pl.when(t == NSUB - 1)
        def _():
            pltpu.sync_copy(s_sh.at[pl.ds(15 * 632, 520)],
                            w1_v.at[pl.ds(0, 520)])
            pltpu.sync_copy(w1_v.at[pl.ds(0, 520)],
                            s_hbm.at[pl.ds(15 * 632, 520)])


def _sc2_body(src_hbm, dst_hbm, w_all_hbm, feat_hbm,
              acc_hbm,
              srcc_v, dstc_v, w1_v, rows_v, wb_v, acc_sh, sem):
    c = lax.axis_index("c")
    t = lax.axis_index("s")

    _zero_vmem(wb_v, RSTRIPE)
    pltpu.sync_copy(wb_v, acc_sh.at[pl.ds(t * RSTRIPE, RSTRIPE)])
    plsc.subcore_barrier()

    own_base = (t * 2 + c) * EPT
    pltpu.sync_copy(w_all_hbm.at[pl.ds(own_base, EPT)], w1_v)

    _p2_chunks(own_base, src_hbm, dst_hbm, srcc_v, dstc_v, w1_v, rows_v,
               feat_hbm, acc_sh, sem)

    plsc.subcore_barrier()
    r0 = t * RSTRIPE
    pltpu.sync_copy(acc_sh.at[pl.ds(r0, RSTRIPE)], wb_v)
    pltpu.sync_copy(wb_v, acc_hbm.at[c, pl.ds(r0, RSTRIPE)])


_SC_MESH = plsc.VectorSubcoreMesh(core_axis_name="c", subcore_axis_name="s")
_SC_PARAMS = pltpu.CompilerParams(needs_layout_passes=False, use_tc_tiling_on_sc=False)

_sc1 = pl.kernel(
    _sc1_body,
    compiler_params=_SC_PARAMS,
    out_type=[
        jax.ShapeDtypeStruct((E,), jnp.float32),
        jax.ShapeDtypeStruct((N,), jnp.float32),
        jax.ShapeDtypeStruct((NCORE, N, D_LAT), jnp.float32),
    ],
    mesh=_SC_MESH,
    scratch_types=[
        pltpu.VMEM((N,), jnp.float32),
        pltpu.VMEM((N,), jnp.float32),
        pltpu.VMEM((CH,), jnp.int32),
        pltpu.VMEM((CH,), jnp.int32),
        pltpu.VMEM((EPT,), jnp.float32),
        pltpu.VMEM((CH, D_LAT), jnp.float32),
        pltpu.VMEM((RSTRIPE, D_LAT), jnp.float32),
        pltpu.VMEM_SHARED((N,), jnp.float32),
        pltpu.VMEM_SHARED((N, D_LAT), jnp.float32),
        pltpu.SemaphoreType.DMA,
    ],
)

_sc2 = pl.kernel(
    _sc2_body,
    compiler_params=_SC_PARAMS,
    out_type=jax.ShapeDtypeStruct((NCORE, N, D_LAT), jnp.float32),
    mesh=_SC_MESH,
    scratch_types=[
        pltpu.VMEM((CH,), jnp.int32),
        pltpu.VMEM((CH,), jnp.int32),
        pltpu.VMEM((EPT,), jnp.float32),
        pltpu.VMEM((CH, D_LAT), jnp.float32),
        pltpu.VMEM((RSTRIPE, D_LAT), jnp.float32),
        pltpu.VMEM_SHARED((N, D_LAT), jnp.float32),
        pltpu.SemaphoreType.DMA,
    ],
)


def kernel(x, W1, S1, att_src1, att_dst1, W2, graph_edges):
    src = graph_edges[0]
    dst = graph_edges[1]

    h_src, a_src, a_dst = _enc_stage(x, W1, S1, att_src1, att_dst1)
    w_all, s, acc1 = _sc1(src, dst, a_src.reshape(N), a_dst.reshape(N),
                          h_src)
    s2 = s.reshape(N, 1)
    emb, hd = _mid_stage(acc1[0], acc1[1], s2, W2)
    acc2 = _sc2(src, dst, w_all, hd)
    rec = _dec_stage(acc2[0], acc2[1], s2, W1)
    return emb, rec


# speedup vs baseline: 38.9657x; 2.9131x over previous
import jax
import jax.numpy as jnp
from jax import lax
from jax.experimental import pallas as pl
from jax.experimental.pallas import tpu as pltpu
from jax.experimental.pallas import tpu_sc as plsc

N = 10000
E = 320000
D_IN, D_LAT, D_EMB = 128, 64, 32

NCORE, NSUB = 2, 16
CH = 80
NCH = 125
EPT = CH * NCH
RSTRIPE = N // NSUB
TBLK = 1000


def _enc_body(x_ref, w1_ref, s1_ref, avs_ref, avd_ref, hs_ref, as_ref, ad_ref):
    xb = x_ref[...]
    hs = jnp.dot(xb, w1_ref[...], preferred_element_type=jnp.float32)
    hd = jnp.dot(xb, s1_ref[...], preferred_element_type=jnp.float32)
    hs_ref[...] = hs
    as_ref[...] = jnp.sum(hs * avs_ref[...][None, :], axis=1)[:, None]
    ad_ref[...] = jnp.sum(hd * avd_ref[...][None, :], axis=1)[:, None]


def _enc_stage(x, W1, S1, avs, avd):
    grid = (N // TBLK,)
    return pl.pallas_call(
        _enc_body,
        grid=grid,
        in_specs=[
            pl.BlockSpec((TBLK, D_IN), lambda i: (i, 0)),
            pl.BlockSpec((D_IN, D_LAT), lambda i: (0, 0)),
            pl.BlockSpec((D_IN, D_LAT), lambda i: (0, 0)),
            pl.BlockSpec((D_LAT,), lambda i: (0,)),
            pl.BlockSpec((D_LAT,), lambda i: (0,)),
        ],
        out_specs=[
            pl.BlockSpec((TBLK, D_LAT), lambda i: (i, 0)),
            pl.BlockSpec((TBLK, 1), lambda i: (i, 0)),
            pl.BlockSpec((TBLK, 1), lambda i: (i, 0)),
        ],
        out_shape=[
            jax.ShapeDtypeStruct((N, D_LAT), jnp.float32),
            jax.ShapeDtypeStruct((N, 1), jnp.float32),
            jax.ShapeDtypeStruct((N, 1), jnp.float32),
        ],
    )(x, W1, S1, avs, avd)


def _mid_body(p0_ref, p1_ref, s0_ref, s1_ref, w2_ref, emb_ref):
    t = (p0_ref[...] + p1_ref[...]) / (s0_ref[...] + s1_ref[...] + 1e-16)
    h1 = jnp.where(t > 0, t, jnp.exp(t) - 1.0)
    emb_ref[...] = jnp.dot(h1, w2_ref[...],
                           preferred_element_type=jnp.float32)


def _mid_stage(p0, p1, s0, s1, W2):
    grid = (N // TBLK,)
    return pl.pallas_call(
        _mid_body,
        grid=grid,
        in_specs=[
            pl.BlockSpec((TBLK, D_LAT), lambda i: (i, 0)),
            pl.BlockSpec((TBLK, D_LAT), lambda i: (i, 0)),
            pl.BlockSpec((TBLK, 1), lambda i: (i, 0)),
            pl.BlockSpec((TBLK, 1), lambda i: (i, 0)),
            pl.BlockSpec((D_LAT, D_EMB), lambda i: (0, 0)),
        ],
        out_specs=pl.BlockSpec((TBLK, D_EMB), lambda i: (i, 0)),
        out_shape=jax.ShapeDtypeStruct((N, D_EMB), jnp.float32),
    )(p0, p1, s0, s1, W2)


def _dec_body(p0_ref, p1_ref, s0_ref, s1_ref, w2_ref, w1_ref, rec_ref):
    u = (p0_ref[...] + p1_ref[...]) / (s0_ref[...] + s1_ref[...] + 1e-16)
    t = lax.dot_general(u, w2_ref[...], (((1,), (1,)), ((), ())),
                        preferred_element_type=jnp.float32)
    d1 = jnp.where(t > 0, t, jnp.exp(t) - 1.0)
    rec_ref[...] = lax.dot_general(d1, w1_ref[...], (((1,), (1,)), ((), ())),
                                   preferred_element_type=jnp.float32)


def _dec_stage(p0, p1, s0, s1, W2, W1):
    grid = (N // TBLK,)
    return pl.pallas_call(
        _dec_body,
        grid=grid,
        in_specs=[
            pl.BlockSpec((TBLK, D_EMB), lambda i: (i, 0)),
            pl.BlockSpec((TBLK, D_EMB), lambda i: (i, 0)),
            pl.BlockSpec((TBLK, 1), lambda i: (i, 0)),
            pl.BlockSpec((TBLK, 1), lambda i: (i, 0)),
            pl.BlockSpec((D_LAT, D_EMB), lambda i: (0, 0)),
            pl.BlockSpec((D_IN, D_LAT), lambda i: (0, 0)),
        ],
        out_specs=pl.BlockSpec((TBLK, D_IN), lambda i: (i, 0)),
        out_shape=jax.ShapeDtypeStruct((N, D_IN), jnp.float32),
    )(p0, p1, s0, s1, W2, W1)


def _zero_vmem(ref, nrow):
    z = jnp.zeros((16,), jnp.float32)

    def body(r, _):
        for q in range(ref.shape[1] // 16):
            ref[r, pl.ds(q * 16, 16)] = z
        return 0

    lax.fori_loop(0, nrow, body, 0)


def _p2_chunks(src2_v, dst2_v, w1_v, rows_a, rows_b, feat_hbm, acc_sh,
               sem_a, sem_b, d):
    def issue(ch, rows, sem):
        pltpu.async_copy(feat_hbm.at[src2_v.at[ch]], rows, sem)

    def wait(ch, rows, sem):
        pltpu.make_async_copy(feat_hbm.at[src2_v.at[ch]], rows, sem).wait()

    def scale_scatter(ch, rows):
        for g in range(CH // 16):
            wvec = w1_v[pl.ds(ch * CH + g * 16, 16)]
            for k in range(16):
                wv = jnp.full((16,), wvec[k])
                e = g * 16 + k
                for q in range(d // 16):
                    rows[e, pl.ds(q * 16, 16)] = (
                        rows[e, pl.ds(q * 16, 16)] * wv)
        pltpu.sync_copy(rows, acc_sh.at[dst2_v.at[ch]], add=True)

    issue(0, rows_a, sem_a)

    def pair(g, _):
        ch0 = 2 * g
        issue(ch0 + 1, rows_b, sem_b)
        wait(ch0, rows_a, sem_a)
        scale_scatter(ch0, rows_a)
        issue(ch0 + 2, rows_a, sem_a)
        wait(ch0 + 1, rows_b, sem_b)
        scale_scatter(ch0 + 1, rows_b)
        return 0

    lax.fori_loop(0, (NCH - 1) // 2, pair, 0)
    wait(NCH - 1, rows_a, sem_a)
    scale_scatter(NCH - 1, rows_a)


def _sc1_body(src3_hbm, dst3_hbm, asrc_hbm, adst_hbm, hsrc_hbm,
              w_hbm, s_hbm, acc_hbm,
              asrc_v, adst_v, src2_v, dst2_v, w1_v, rows_a, rows_b, zb_v,
              s_sh, acc_sh, sem_a, sem_b):
    c = lax.axis_index("c")
    t = lax.axis_index("s")

    pltpu.sync_copy(asrc_hbm, asrc_v)
    pltpu.sync_copy(adst_hbm, adst_v)

    _zero_vmem(zb_v, NCH)
    for k5 in range(RSTRIPE // NCH):
        pltpu.sync_copy(zb_v, acc_sh.at[pl.ds(t * RSTRIPE + k5 * NCH, NCH)])

    def zs(j, _):
        w1_v[pl.ds(j * 16, 16)] = jnp.zeros((16,), jnp.float32)
        return 0

    lax.fori_loop(0, 40, zs, 0)

    @pl.when(t < NSUB - 1)
    def _():
        pltpu.sync_copy(w1_v.at[pl.ds(0, 632)], s_sh.at[pl.ds(t * 632, 632)])

    @pl.when(t == NSUB - 1)
    def _():
        pltpu.sync_copy(w1_v.at[pl.ds(0, 520)], s_sh.at[pl.ds(15 * 632, 520)])

    plsc.subcore_barrier()

    wid = t * 2 + c
    pltpu.sync_copy(src3_hbm.at[pl.ds(wid * NCH, NCH)], src2_v)
    pltpu.sync_copy(dst3_hbm.at[pl.ds(wid * NCH, NCH)], dst2_v)

    def chunk(ch, _):
        for j in range(CH // 16):
            s16 = src2_v[ch, pl.ds(j * 16, 16)]
            d16 = dst2_v[ch, pl.ds(j * 16, 16)]
            a_s = plsc.load_gather(asrc_v, [s16])
            a_d = plsc.load_gather(adst_v, [d16])
            z = a_s + a_d
            sg = 1.0 / (1.0 + jnp.exp(-z))
            w1_v[pl.ds(ch * CH + j * 16, 16)] = jnp.exp(sg)
        pltpu.sync_copy(w1_v.at[pl.ds(ch * CH, CH)],
                        s_sh.at[dst2_v.at[ch]], add=True)
        return 0

    lax.fori_loop(0, NCH, chunk, 0)
    pltpu.sync_copy(w1_v, w_hbm.at[pl.ds(wid * EPT, EPT)])

    _p2_chunks(src2_v, dst2_v, w1_v, rows_a, rows_b, hsrc_hbm, acc_sh,
               sem_a, sem_b, D_LAT)

    plsc.subcore_barrier()

    r0 = t * RSTRIPE
    pltpu.sync_copy(acc_sh.at[pl.ds(r0, RSTRIPE)],
                    acc_hbm.at[c, pl.ds(r0, RSTRIPE)])

    @pl.when(t < NSUB - 1)
    def _():
        pltpu.sync_copy(s_sh.at[pl.ds(t * 632, 632)],
                        s_hbm.at[c, pl.ds(t * 632, 632)])

    @pl.when(t == NSUB - 1)
    def _():
        pltpu.sync_copy(s_sh.at[pl.ds(15 * 632, 520)],
                        s_hbm.at[c, pl.ds(15 * 632, 520)])


def _sc2_body(src3_hbm, dst3_hbm, w_all_hbm, feat_hbm,
              acc_hbm,
              src2_v, dst2_v, w1_v, rows_a, rows_b, zb_v, acc_sh,
              sem_a, sem_b):
    c = lax.axis_index("c")
    t = lax.axis_index("s")

    _zero_vmem(zb_v, NCH)
    for k5 in range(RSTRIPE // NCH):
        pltpu.sync_copy(zb_v, acc_sh.at[pl.ds(t * RSTRIPE + k5 * NCH, NCH)])
    plsc.subcore_barrier()

    wid = t * 2 + c
    pltpu.sync_copy(src3_hbm.at[pl.ds(wid * NCH, NCH)], src2_v)
    pltpu.sync_copy(dst3_hbm.at[pl.ds(wid * NCH, NCH)], dst2_v)
    pltpu.sync_copy(w_all_hbm.at[pl.ds(wid * EPT, EPT)], w1_v)

    _p2_chunks(src2_v, dst2_v, w1_v, rows_a, rows_b, feat_hbm, acc_sh,
               sem_a, sem_b, D_EMB)

    plsc.subcore_barrier()
    r0 = t * RSTRIPE
    pltpu.sync_copy(acc_sh.at[pl.ds(r0, RSTRIPE)],
                    acc_hbm.at[c, pl.ds(r0, RSTRIPE)])


_SC_MESH = plsc.VectorSubcoreMesh(core_axis_name="c", subcore_axis_name="s")
_SC_PARAMS = pltpu.CompilerParams(needs_layout_passes=False,
                                  use_tc_tiling_on_sc=False)

_sc1 = pl.kernel(
    _sc1_body,
    compiler_params=_SC_PARAMS,
    out_type=[
        jax.ShapeDtypeStruct((E,), jnp.float32),
        jax.ShapeDtypeStruct((NCORE, N), jnp.float32),
        jax.ShapeDtypeStruct((NCORE, N, D_LAT), jnp.float32),
    ],
    mesh=_SC_MESH,
    scratch_types=[
        pltpu.VMEM((N,), jnp.float32),
        pltpu.VMEM((N,), jnp.float32),
        pltpu.VMEM((NCH, CH), jnp.int32),
        pltpu.VMEM((NCH, CH), jnp.int32),
        pltpu.VMEM((EPT,), jnp.float32),
        pltpu.VMEM((CH, D_LAT), jnp.float32),
        pltpu.VMEM((CH, D_LAT), jnp.float32),
        pltpu.VMEM((NCH, D_LAT), jnp.float32),
        pltpu.VMEM_SHARED((N,), jnp.float32),
        pltpu.VMEM_SHARED((N, D_LAT), jnp.float32),
        pltpu.SemaphoreType.DMA,
        pltpu.SemaphoreType.DMA,
    ],
)

_sc2 = pl.kernel(
    _sc2_body,
    compiler_params=_SC_PARAMS,
    out_type=jax.ShapeDtypeStruct((NCORE, N, D_EMB), jnp.float32),
    mesh=_SC_MESH,
    scratch_types=[
        pltpu.VMEM((NCH, CH), jnp.int32),
        pltpu.VMEM((NCH, CH), jnp.int32),
        pltpu.VMEM((EPT,), jnp.float32),
        pltpu.VMEM((CH, D_EMB), jnp.float32),
        pltpu.VMEM((CH, D_EMB), jnp.float32),
        pltpu.VMEM((NCH, D_EMB), jnp.float32),
        pltpu.VMEM_SHARED((N, D_EMB), jnp.float32),
        pltpu.SemaphoreType.DMA,
        pltpu.SemaphoreType.DMA,
    ],
)


def kernel(x, W1, S1, att_src1, att_dst1, W2, graph_edges):
    src3 = graph_edges[0].reshape(E // CH, CH)
    dst3 = graph_edges[1].reshape(E // CH, CH)

    h_src, a_src, a_dst = _enc_stage(x, W1, S1, att_src1, att_dst1)
    w_all, s, acc1 = _sc1(src3, dst3, a_src.reshape(N), a_dst.reshape(N),
                          h_src)
    s0 = s[0].reshape(N, 1)
    s1 = s[1].reshape(N, 1)
    emb = _mid_stage(acc1[0], acc1[1], s0, s1, W2)
    acc2 = _sc2(src3, dst3, w_all, emb)
    rec = _dec_stage(acc2[0], acc2[1], s0, s1, W2, W1)
    return emb, rec


# speedup vs baseline: 45.1203x; 1.1580x over previous
import jax
import jax.numpy as jnp
from jax import lax
from jax.experimental import pallas as pl
from jax.experimental.pallas import tpu as pltpu
from jax.experimental.pallas import tpu_sc as plsc

N = 10000
E = 320000
D_IN, D_LAT, D_EMB = 128, 64, 32

NCORE, NSUB = 2, 16
CH = 80
NCH = 125
EPT = CH * NCH
RSTRIPE = N // NSUB
TBLK = 1000


def _enc_body(x_ref, w1_ref, s1_ref, avs_ref, avd_ref, hs_ref, as_ref, ad_ref):
    xb = x_ref[...]
    hs = jnp.dot(xb, w1_ref[...], preferred_element_type=jnp.float32)
    hd = jnp.dot(xb, s1_ref[...], preferred_element_type=jnp.float32)
    hs_ref[...] = hs
    as_ref[...] = jnp.sum(hs * avs_ref[...][None, :], axis=1)[:, None]
    ad_ref[...] = jnp.sum(hd * avd_ref[...][None, :], axis=1)[:, None]


def _enc_stage(x, W1, S1, avs, avd):
    grid = (N // TBLK,)
    return pl.pallas_call(
        _enc_body,
        grid=grid,
        in_specs=[
            pl.BlockSpec((TBLK, D_IN), lambda i: (i, 0)),
            pl.BlockSpec((D_IN, D_LAT), lambda i: (0, 0)),
            pl.BlockSpec((D_IN, D_LAT), lambda i: (0, 0)),
            pl.BlockSpec((D_LAT,), lambda i: (0,)),
            pl.BlockSpec((D_LAT,), lambda i: (0,)),
        ],
        out_specs=[
            pl.BlockSpec((TBLK, D_LAT), lambda i: (i, 0)),
            pl.BlockSpec((TBLK, 1), lambda i: (i, 0)),
            pl.BlockSpec((TBLK, 1), lambda i: (i, 0)),
        ],
        out_shape=[
            jax.ShapeDtypeStruct((N, D_LAT), jnp.float32),
            jax.ShapeDtypeStruct((N, 1), jnp.float32),
            jax.ShapeDtypeStruct((N, 1), jnp.float32),
        ],
    )(x, W1, S1, avs, avd)


def _mid_body(p0_ref, p1_ref, s0_ref, s1_ref, w2_ref, emb_ref):
    t = (p0_ref[...] + p1_ref[...]) / (s0_ref[...] + s1_ref[...] + 1e-16)
    h1 = jnp.where(t > 0, t, jnp.exp(t) - 1.0)
    emb_ref[...] = jnp.dot(h1, w2_ref[...],
                           preferred_element_type=jnp.float32)


def _mid_stage(p0, p1, s0, s1, W2):
    grid = (N // TBLK,)
    return pl.pallas_call(
        _mid_body,
        grid=grid,
        in_specs=[
            pl.BlockSpec((TBLK, D_LAT), lambda i: (i, 0)),
            pl.BlockSpec((TBLK, D_LAT), lambda i: (i, 0)),
            pl.BlockSpec((TBLK, 1), lambda i: (i, 0)),
            pl.BlockSpec((TBLK, 1), lambda i: (i, 0)),
            pl.BlockSpec((D_LAT, D_EMB), lambda i: (0, 0)),
        ],
        out_specs=pl.BlockSpec((TBLK, D_EMB), lambda i: (i, 0)),
        out_shape=jax.ShapeDtypeStruct((N, D_EMB), jnp.float32),
    )(p0, p1, s0, s1, W2)


def _dec_body(p0_ref, p1_ref, s0_ref, s1_ref, w2_ref, w1_ref, rec_ref):
    u = (p0_ref[...] + p1_ref[...]) / (s0_ref[...] + s1_ref[...] + 1e-16)
    t = lax.dot_general(u, w2_ref[...], (((1,), (1,)), ((), ())),
                        preferred_element_type=jnp.float32)
    d1 = jnp.where(t > 0, t, jnp.exp(t) - 1.0)
    rec_ref[...] = lax.dot_general(d1, w1_ref[...], (((1,), (1,)), ((), ())),
                                   preferred_element_type=jnp.float32)


def _dec_stage(p0, p1, s0, s1, W2, W1):
    grid = (N // TBLK,)
    return pl.pallas_call(
        _dec_body,
        grid=grid,
        in_specs=[
            pl.BlockSpec((TBLK, D_EMB), lambda i: (i, 0)),
            pl.BlockSpec((TBLK, D_EMB), lambda i: (i, 0)),
            pl.BlockSpec((TBLK, 1), lambda i: (i, 0)),
            pl.BlockSpec((TBLK, 1), lambda i: (i, 0)),
            pl.BlockSpec((D_LAT, D_EMB), lambda i: (0, 0)),
            pl.BlockSpec((D_IN, D_LAT), lambda i: (0, 0)),
        ],
        out_specs=pl.BlockSpec((TBLK, D_IN), lambda i: (i, 0)),
        out_shape=jax.ShapeDtypeStruct((N, D_IN), jnp.float32),
    )(p0, p1, s0, s1, W2, W1)


def _zero_vmem(ref, nrow):
    z = jnp.zeros((16,), jnp.float32)

    def body(r, _):
        for q in range(ref.shape[1] // 16):
            ref[r, pl.ds(q * 16, 16)] = z
        return 0

    lax.fori_loop(0, nrow, body, 0)


NB = 5


def _p2_ring(src2_v, dst2_v, w1_v, rows, gsems, ssems, feat_hbm, acc_sh, d):
    def g_issue(ch, b):
        pltpu.async_copy(feat_hbm.at[src2_v.at[ch]], rows[b], gsems[b])

    def g_wait(b):
        pltpu.make_async_copy(feat_hbm.at[src2_v.at[0]], rows[b],
                              gsems[b]).wait()

    def s_issue(ch, b):
        pltpu.async_copy(rows[b], acc_sh.at[dst2_v.at[ch]], ssems[b],
                         add=True)

    def s_wait(b):
        pltpu.make_async_copy(rows[b], acc_sh.at[dst2_v.at[0]],
                              ssems[b]).wait()

    def scale(ch, b):
        for g in range(CH // 16):
            wvec = w1_v[pl.ds(ch * CH + g * 16, 16)]
            for k in range(16):
                wv = jnp.full((16,), wvec[k])
                e = g * 16 + k
                for q in range(d // 16):
                    rows[b][e, pl.ds(q * 16, 16)] = (
                        rows[b][e, pl.ds(q * 16, 16)] * wv)

    g_issue(0, 0)
    g_issue(1, 1)

    def group(g, _):
        for b in range(NB):
            ch = g * NB + b

            @pl.when(ch + 2 < NCH)
            def _():
                nb = (b + 2) % NB

                @pl.when(ch >= 3)
                def _():
                    s_wait(nb)

                g_issue(ch + 2, nb)

            g_wait(b)
            scale(ch, b)
            s_issue(ch, b)
        return 0

    lax.fori_loop(0, NCH // NB, group, 0)
    for b in range(NB):
        s_wait(b)


def _sc1_body(src3_hbm, dst3_hbm, asrc_hbm, adst_hbm, hsrc_hbm,
              w_hbm, s_hbm, acc_hbm,
              asrc_v, adst_v, src2_v, dst2_v, w1_v,
              r0_v, r1_v, r2_v, r3_v, r4_v, zb_v,
              s_sh, acc_sh,
              g0, g1, g2, g3, g4, s0, s1, s2, s3, s4, sem_s):
    c = lax.axis_index("c")
    t = lax.axis_index("s")

    pltpu.sync_copy(asrc_hbm, asrc_v)
    pltpu.sync_copy(adst_hbm, adst_v)

    _zero_vmem(zb_v, NCH)
    for k5 in range(RSTRIPE // NCH):
        pltpu.sync_copy(zb_v, acc_sh.at[pl.ds(t * RSTRIPE + k5 * NCH, NCH)])

    def zs(j, _):
        w1_v[pl.ds(j * 16, 16)] = jnp.zeros((16,), jnp.float32)
        return 0

    lax.fori_loop(0, 40, zs, 0)

    @pl.when(t < NSUB - 1)
    def _():
        pltpu.sync_copy(w1_v.at[pl.ds(0, 632)], s_sh.at[pl.ds(t * 632, 632)])

    @pl.when(t == NSUB - 1)
    def _():
        pltpu.sync_copy(w1_v.at[pl.ds(0, 520)], s_sh.at[pl.ds(15 * 632, 520)])

    plsc.subcore_barrier()

    wid = t * 2 + c
    pltpu.sync_copy(src3_hbm.at[pl.ds(wid * NCH, NCH)], src2_v)
    pltpu.sync_copy(dst3_hbm.at[pl.ds(wid * NCH, NCH)], dst2_v)

    def chunk(ch, _):
        for j in range(CH // 16):
            s16 = src2_v[ch, pl.ds(j * 16, 16)]
            d16 = dst2_v[ch, pl.ds(j * 16, 16)]
            a_s = plsc.load_gather(asrc_v, [s16])
            a_d = plsc.load_gather(adst_v, [d16])
            z = a_s + a_d
            sg = 1.0 / (1.0 + jnp.exp(-z))
            w1_v[pl.ds(ch * CH + j * 16, 16)] = jnp.exp(sg)
        pltpu.async_copy(w1_v.at[pl.ds(ch * CH, CH)],
                         s_sh.at[dst2_v.at[ch]], sem_s, add=True)
        return 0

    lax.fori_loop(0, NCH, chunk, 0)
    pltpu.sync_copy(w1_v, w_hbm.at[pl.ds(wid * EPT, EPT)])

    _p2_ring(src2_v, dst2_v, w1_v, [r0_v, r1_v, r2_v, r3_v, r4_v],
             [g0, g1, g2, g3, g4], [s0, s1, s2, s3, s4],
             hsrc_hbm, acc_sh, D_LAT)

    def p1_drain(i, _):
        pltpu.make_async_copy(w1_v.at[pl.ds(0, CH)],
                              s_sh.at[dst2_v.at[0]], sem_s).wait()
        return 0

    lax.fori_loop(0, NCH, p1_drain, 0)

    plsc.subcore_barrier()

    r0 = t * RSTRIPE
    pltpu.sync_copy(acc_sh.at[pl.ds(r0, RSTRIPE)],
                    acc_hbm.at[c, pl.ds(r0, RSTRIPE)])

    @pl.when(t < NSUB - 1)
    def _():
        pltpu.sync_copy(s_sh.at[pl.ds(t * 632, 632)],
                        s_hbm.at[c, pl.ds(t * 632, 632)])

    @pl.when(t == NSUB - 1)
    def _():
        pltpu.sync_copy(s_sh.at[pl.ds(15 * 632, 520)],
                        s_hbm.at[c, pl.ds(15 * 632, 520)])


def _sc2_body(src3_hbm, dst3_hbm, w_all_hbm, feat_hbm,
              acc_hbm,
              src2_v, dst2_v, w1_v,
              r0_v, r1_v, r2_v, r3_v, r4_v, zb_v, acc_sh,
              g0, g1, g2, g3, g4, s0, s1, s2, s3, s4):
    c = lax.axis_index("c")
    t = lax.axis_index("s")

    _zero_vmem(zb_v, NCH)
    for k5 in range(RSTRIPE // NCH):
        pltpu.sync_copy(zb_v, acc_sh.at[pl.ds(t * RSTRIPE + k5 * NCH, NCH)])
    plsc.subcore_barrier()

    wid = t * 2 + c
    pltpu.sync_copy(src3_hbm.at[pl.ds(wid * NCH, NCH)], src2_v)
    pltpu.sync_copy(dst3_hbm.at[pl.ds(wid * NCH, NCH)], dst2_v)
    pltpu.sync_copy(w_all_hbm.at[pl.ds(wid * EPT, EPT)], w1_v)

    _p2_ring(src2_v, dst2_v, w1_v, [r0_v, r1_v, r2_v, r3_v, r4_v],
             [g0, g1, g2, g3, g4], [s0, s1, s2, s3, s4],
             feat_hbm, acc_sh, D_EMB)

    plsc.subcore_barrier()
    r0 = t * RSTRIPE
    pltpu.sync_copy(acc_sh.at[pl.ds(r0, RSTRIPE)],
                    acc_hbm.at[c, pl.ds(r0, RSTRIPE)])


_SC_MESH = plsc.VectorSubcoreMesh(core_axis_name="c", subcore_axis_name="s")
_SC_PARAMS = pltpu.CompilerParams(needs_layout_passes=False,
                                  use_tc_tiling_on_sc=False)

_sc1 = pl.kernel(
    _sc1_body,
    compiler_params=_SC_PARAMS,
    out_type=[
        jax.ShapeDtypeStruct((E,), jnp.float32),
        jax.ShapeDtypeStruct((NCORE, N), jnp.float32),
        jax.ShapeDtypeStruct((NCORE, N, D_LAT), jnp.float32),
    ],
    mesh=_SC_MESH,
    scratch_types=[
        pltpu.VMEM((N,), jnp.float32),
        pltpu.VMEM((N,), jnp.float32),
        pltpu.VMEM((NCH, CH), jnp.int32),
        pltpu.VMEM((NCH, CH), jnp.int32),
        pltpu.VMEM((EPT,), jnp.float32),
        pltpu.VMEM((CH, D_LAT), jnp.float32),
        pltpu.VMEM((CH, D_LAT), jnp.float32),
        pltpu.VMEM((CH, D_LAT), jnp.float32),
        pltpu.VMEM((CH, D_LAT), jnp.float32),
        pltpu.VMEM((CH, D_LAT), jnp.float32),
        pltpu.VMEM((NCH, D_LAT), jnp.float32),
        pltpu.VMEM_SHARED((N,), jnp.float32),
        pltpu.VMEM_SHARED((N, D_LAT), jnp.float32),
        pltpu.SemaphoreType.DMA,
        pltpu.SemaphoreType.DMA,
        pltpu.SemaphoreType.DMA,
        pltpu.SemaphoreType.DMA,
        pltpu.SemaphoreType.DMA,
        pltpu.SemaphoreType.DMA,
        pltpu.SemaphoreType.DMA,
        pltpu.SemaphoreType.DMA,
        pltpu.SemaphoreType.DMA,
        pltpu.SemaphoreType.DMA,
        pltpu.SemaphoreType.DMA,
    ],
)

_sc2 = pl.kernel(
    _sc2_body,
    compiler_params=_SC_PARAMS,
    out_type=jax.ShapeDtypeStruct((NCORE, N, D_EMB), jnp.float32),
    mesh=_SC_MESH,
    scratch_types=[
        pltpu.VMEM((NCH, CH), jnp.int32),
        pltpu.VMEM((NCH, CH), jnp.int32),
        pltpu.VMEM((EPT,), jnp.float32),
        pltpu.VMEM((CH, D_EMB), jnp.float32),
        pltpu.VMEM((CH, D_EMB), jnp.float32),
        pltpu.VMEM((CH, D_EMB), jnp.float32),
        pltpu.VMEM((CH, D_EMB), jnp.float32),
        pltpu.VMEM((CH, D_EMB), jnp.float32),
        pltpu.VMEM((NCH, D_EMB), jnp.float32),
        pltpu.VMEM_SHARED((N, D_EMB), jnp.float32),
        pltpu.SemaphoreType.DMA,
        pltpu.SemaphoreType.DMA,
        pltpu.SemaphoreType.DMA,
        pltpu.SemaphoreType.DMA,
        pltpu.SemaphoreType.DMA,
        pltpu.SemaphoreType.DMA,
        pltpu.SemaphoreType.DMA,
        pltpu.SemaphoreType.DMA,
        pltpu.SemaphoreType.DMA,
        pltpu.SemaphoreType.DMA,
    ],
)


def kernel(x, W1, S1, att_src1, att_dst1, W2, graph_edges):
    src3 = graph_edges[0].reshape(E // CH, CH)
    dst3 = graph_edges[1].reshape(E // CH, CH)

    h_src, a_src, a_dst = _enc_stage(x, W1, S1, att_src1, att_dst1)
    w_all, s, acc1 = _sc1(src3, dst3, a_src.reshape(N), a_dst.reshape(N),
                          h_src)
    s0 = s[0].reshape(N, 1)
    s1 = s[1].reshape(N, 1)
    emb = _mid_stage(acc1[0], acc1[1], s0, s1, W2)
    acc2 = _sc2(src3, dst3, w_all, emb)
    rec = _dec_stage(acc2[0], acc2[1], s0, s1, W2, W1)
    return emb, rec


# speedup vs baseline: 46.9949x; 1.0415x over previous
import jax
import jax.numpy as jnp
from jax import lax
from jax.experimental import pallas as pl
from jax.experimental.pallas import tpu as pltpu
from jax.experimental.pallas import tpu_sc as plsc

N = 10000
E = 320000
D_IN, D_LAT, D_EMB = 128, 64, 32

NCORE, NSUB = 2, 16
CH = 80
NCH = 125
EPT = CH * NCH
RSTRIPE = N // NSUB
TBLK = 1000


def _enc_body(x_ref, w1_ref, s1_ref, avs_ref, avd_ref, hs_ref, as_ref, ad_ref):
    xb = x_ref[...]
    hs = jnp.dot(xb, w1_ref[...], preferred_element_type=jnp.float32)
    hd = jnp.dot(xb, s1_ref[...], preferred_element_type=jnp.float32)
    hs_ref[...] = hs
    as_ref[...] = jnp.sum(hs * avs_ref[...][None, :], axis=1)[:, None]
    ad_ref[...] = jnp.sum(hd * avd_ref[...][None, :], axis=1)[:, None]


def _enc_stage(x, W1, S1, avs, avd):
    grid = (N // TBLK,)
    return pl.pallas_call(
        _enc_body,
        grid=grid,
        in_specs=[
            pl.BlockSpec((TBLK, D_IN), lambda i: (i, 0)),
            pl.BlockSpec((D_IN, D_LAT), lambda i: (0, 0)),
            pl.BlockSpec((D_IN, D_LAT), lambda i: (0, 0)),
            pl.BlockSpec((D_LAT,), lambda i: (0,)),
            pl.BlockSpec((D_LAT,), lambda i: (0,)),
        ],
        out_specs=[
            pl.BlockSpec((TBLK, D_LAT), lambda i: (i, 0)),
            pl.BlockSpec((TBLK, 1), lambda i: (i, 0)),
            pl.BlockSpec((TBLK, 1), lambda i: (i, 0)),
        ],
        out_shape=[
            jax.ShapeDtypeStruct((N, D_LAT), jnp.float32),
            jax.ShapeDtypeStruct((N, 1), jnp.float32),
            jax.ShapeDtypeStruct((N, 1), jnp.float32),
        ],
    )(x, W1, S1, avs, avd)


def _mid_body(p0_ref, p1_ref, s0_ref, s1_ref, w2_ref, emb_ref):
    t = (p0_ref[...] + p1_ref[...]) / (s0_ref[...] + s1_ref[...] + 1e-16)
    h1 = jnp.where(t > 0, t, jnp.exp(t) - 1.0)
    emb_ref[...] = jnp.dot(h1, w2_ref[...],
                           preferred_element_type=jnp.float32)


def _mid_stage(p0, p1, s0, s1, W2):
    grid = (N // TBLK,)
    return pl.pallas_call(
        _mid_body,
        grid=grid,
        in_specs=[
            pl.BlockSpec((TBLK, D_LAT), lambda i: (i, 0)),
            pl.BlockSpec((TBLK, D_LAT), lambda i: (i, 0)),
            pl.BlockSpec((TBLK, 1), lambda i: (i, 0)),
            pl.BlockSpec((TBLK, 1), lambda i: (i, 0)),
            pl.BlockSpec((D_LAT, D_EMB), lambda i: (0, 0)),
        ],
        out_specs=pl.BlockSpec((TBLK, D_EMB), lambda i: (i, 0)),
        out_shape=jax.ShapeDtypeStruct((N, D_EMB), jnp.float32),
    )(p0, p1, s0, s1, W2)


def _dec_body(p0_ref, p1_ref, s0_ref, s1_ref, w2_ref, w1_ref, rec_ref):
    u = (p0_ref[...] + p1_ref[...]) / (s0_ref[...] + s1_ref[...] + 1e-16)
    t = lax.dot_general(u, w2_ref[...], (((1,), (1,)), ((), ())),
                        preferred_element_type=jnp.float32)
    d1 = jnp.where(t > 0, t, jnp.exp(t) - 1.0)
    rec_ref[...] = lax.dot_general(d1, w1_ref[...], (((1,), (1,)), ((), ())),
                                   preferred_element_type=jnp.float32)


def _dec_stage(p0, p1, s0, s1, W2, W1):
    grid = (N // TBLK,)
    return pl.pallas_call(
        _dec_body,
        grid=grid,
        in_specs=[
            pl.BlockSpec((TBLK, D_EMB), lambda i: (i, 0)),
            pl.BlockSpec((TBLK, D_EMB), lambda i: (i, 0)),
            pl.BlockSpec((TBLK, 1), lambda i: (i, 0)),
            pl.BlockSpec((TBLK, 1), lambda i: (i, 0)),
            pl.BlockSpec((D_LAT, D_EMB), lambda i: (0, 0)),
            pl.BlockSpec((D_IN, D_LAT), lambda i: (0, 0)),
        ],
        out_specs=pl.BlockSpec((TBLK, D_IN), lambda i: (i, 0)),
        out_shape=jax.ShapeDtypeStruct((N, D_IN), jnp.float32),
    )(p0, p1, s0, s1, W2, W1)


def _zero_vmem(ref, nrow):
    z = jnp.zeros((16,), jnp.float32)

    def body(r, _):
        for q in range(ref.shape[1] // 16):
            ref[r, pl.ds(q * 16, 16)] = z
        return 0

    lax.fori_loop(0, nrow, body, 0)


NB = 5


def _p2_ring(src2_v, dst2_v, w1_v, rows, gsems, ssems, feat_hbm, acc_sh, d,
             pre=None):
    def g_issue(ch, b):
        pltpu.async_copy(feat_hbm.at[src2_v.at[ch]], rows[b], gsems[b])

    def g_wait(b):
        pltpu.make_async_copy(feat_hbm.at[src2_v.at[0]], rows[b],
                              gsems[b]).wait()

    def s_issue(ch, b):
        pltpu.async_copy(rows[b], acc_sh.at[dst2_v.at[ch]], ssems[b],
                         add=True)

    def s_wait(b):
        pltpu.make_async_copy(rows[b], acc_sh.at[dst2_v.at[0]],
                              ssems[b]).wait()

    def scale(ch, b):
        for g in range(CH // 16):
            wvec = w1_v[pl.ds(ch * CH + g * 16, 16)]
            for k in range(16):
                wv = jnp.full((16,), wvec[k])
                e = g * 16 + k
                for q in range(d // 16):
                    rows[b][e, pl.ds(q * 16, 16)] = (
                        rows[b][e, pl.ds(q * 16, 16)] * wv)

    g_issue(0, 0)
    g_issue(1, 1)

    def group(g, _):
        for b in range(NB):
            ch = g * NB + b

            @pl.when(ch + 2 < NCH)
            def _():
                nb = (b + 2) % NB

                @pl.when(ch >= 3)
                def _():
                    s_wait(nb)

                g_issue(ch + 2, nb)

            if pre is not None:
                pre(ch)
            g_wait(b)
            scale(ch, b)
            s_issue(ch, b)
        return 0

    lax.fori_loop(0, NCH // NB, group, 0)
    for b in range(NB):
        s_wait(b)


def _sc1_body(src3_hbm, dst3_hbm, asrc_hbm, adst_hbm, hsrc_hbm,
              w_hbm, s_hbm, acc_hbm,
              asrc_v, adst_v, src2_v, dst2_v, w1_v,
              r0_v, r1_v, r2_v, r3_v, r4_v, zb_v,
              s_sh, acc_sh,
              g0, g1, g2, g3, g4, s0, s1, s2, s3, s4, sem_s):
    c = lax.axis_index("c")
    t = lax.axis_index("s")

    pltpu.sync_copy(asrc_hbm, asrc_v)
    pltpu.sync_copy(adst_hbm, adst_v)

    _zero_vmem(zb_v, NCH)
    for k5 in range(RSTRIPE // NCH):
        pltpu.sync_copy(zb_v, acc_sh.at[pl.ds(t * RSTRIPE + k5 * NCH, NCH)])

    def zs(j, _):
        w1_v[pl.ds(j * 16, 16)] = jnp.zeros((16,), jnp.float32)
        return 0

    lax.fori_loop(0, 40, zs, 0)

    @pl.when(t < NSUB - 1)
    def _():
        pltpu.sync_copy(w1_v.at[pl.ds(0, 632)], s_sh.at[pl.ds(t * 632, 632)])

    @pl.when(t == NSUB - 1)
    def _():
        pltpu.sync_copy(w1_v.at[pl.ds(0, 520)], s_sh.at[pl.ds(15 * 632, 520)])

    plsc.subcore_barrier()

    wid = t * 2 + c
    pltpu.sync_copy(src3_hbm.at[pl.ds(wid * NCH, NCH)], src2_v)
    pltpu.sync_copy(dst3_hbm.at[pl.ds(wid * NCH, NCH)], dst2_v)

    def w_chunk(ch):
        for j in range(CH // 16):
            s16 = src2_v[ch, pl.ds(j * 16, 16)]
            d16 = dst2_v[ch, pl.ds(j * 16, 16)]
            a_s = plsc.load_gather(asrc_v, [s16])
            a_d = plsc.load_gather(adst_v, [d16])
            z = a_s + a_d
            sg = 1.0 / (1.0 + jnp.exp(-z))
            w1_v[pl.ds(ch * CH + j * 16, 16)] = jnp.exp(sg)
        pltpu.async_copy(w1_v.at[pl.ds(ch * CH, CH)],
                         s_sh.at[dst2_v.at[ch]], sem_s, add=True)

    _p2_ring(src2_v, dst2_v, w1_v, [r0_v, r1_v, r2_v, r3_v, r4_v],
             [g0, g1, g2, g3, g4], [s0, s1, s2, s3, s4],
             hsrc_hbm, acc_sh, D_LAT, pre=w_chunk)

    pltpu.sync_copy(w1_v, w_hbm.at[pl.ds(wid * EPT, EPT)])

    def p1_drain(i, _):
        pltpu.make_async_copy(w1_v.at[pl.ds(0, CH)],
                              s_sh.at[dst2_v.at[0]], sem_s).wait()
        return 0

    lax.fori_loop(0, NCH, p1_drain, 0)

    plsc.subcore_barrier()

    r0 = t * RSTRIPE
    pltpu.sync_copy(acc_sh.at[pl.ds(r0, RSTRIPE)],
                    acc_hbm.at[c, pl.ds(r0, RSTRIPE)])

    @pl.when(t < NSUB - 1)
    def _():
        pltpu.sync_copy(s_sh.at[pl.ds(t * 632, 632)],
                        s_hbm.at[c, pl.ds(t * 632, 632)])

    @pl.when(t == NSUB - 1)
    def _():
        pltpu.sync_copy(s_sh.at[pl.ds(15 * 632, 520)],
                        s_hbm.at[c, pl.ds(15 * 632, 520)])


def _sc2_body(src3_hbm, dst3_hbm, w_all_hbm, feat_hbm,
              acc_hbm,
              src2_v, dst2_v, w1_v,
              r0_v, r1_v, r2_v, r3_v, r4_v, zb_v, acc_sh,
              g0, g1, g2, g3, g4, s0, s1, s2, s3, s4):
    c = lax.axis_index("c")
    t = lax.axis_index("s")

    _zero_vmem(zb_v, NCH)
    for k5 in range(RSTRIPE // NCH):
        pltpu.sync_copy(zb_v, acc_sh.at[pl.ds(t * RSTRIPE + k5 * NCH, NCH)])
    plsc.subcore_barrier()

    wid = t * 2 + c
    pltpu.sync_copy(src3_hbm.at[pl.ds(wid * NCH, NCH)], src2_v)
    pltpu.sync_copy(dst3_hbm.at[pl.ds(wid * NCH, NCH)], dst2_v)
    pltpu.sync_copy(w_all_hbm.at[pl.ds(wid * EPT, EPT)], w1_v)

    _p2_ring(src2_v, dst2_v, w1_v, [r0_v, r1_v, r2_v, r3_v, r4_v],
             [g0, g1, g2, g3, g4], [s0, s1, s2, s3, s4],
             feat_hbm, acc_sh, D_EMB)

    plsc.subcore_barrier()
    r0 = t * RSTRIPE
    pltpu.sync_copy(acc_sh.at[pl.ds(r0, RSTRIPE)],
                    acc_hbm.at[c, pl.ds(r0, RSTRIPE)])


_SC_MESH = plsc.VectorSubcoreMesh(core_axis_name="c", subcore_axis_name="s")
_SC_PARAMS = pltpu.CompilerParams(needs_layout_passes=False,
                                  use_tc_tiling_on_sc=False)

_sc1 = pl.kernel(
    _sc1_body,
    compiler_params=_SC_PARAMS,
    out_type=[
        jax.ShapeDtypeStruct((E,), jnp.float32),
        jax.ShapeDtypeStruct((NCORE, N), jnp.float32),
        jax.ShapeDtypeStruct((NCORE, N, D_LAT), jnp.float32),
    ],
    mesh=_SC_MESH,
    scratch_types=[
        pltpu.VMEM((N,), jnp.float32),
        pltpu.VMEM((N,), jnp.float32),
        pltpu.VMEM((NCH, CH), jnp.int32),
        pltpu.VMEM((NCH, CH), jnp.int32),
        pltpu.VMEM((EPT,), jnp.float32),
        pltpu.VMEM((CH, D_LAT), jnp.float32),
        pltpu.VMEM((CH, D_LAT), jnp.float32),
        pltpu.VMEM((CH, D_LAT), jnp.float32),
        pltpu.VMEM((CH, D_LAT), jnp.float32),
        pltpu.VMEM((CH, D_LAT), jnp.float32),
        pltpu.VMEM((NCH, D_LAT), jnp.float32),
        pltpu.VMEM_SHARED((N,), jnp.float32),
        pltpu.VMEM_SHARED((N, D_LAT), jnp.float32),
        pltpu.SemaphoreType.DMA,
        pltpu.SemaphoreType.DMA,
        pltpu.SemaphoreType.DMA,
        pltpu.SemaphoreType.DMA,
        pltpu.SemaphoreType.DMA,
        pltpu.SemaphoreType.DMA,
        pltpu.SemaphoreType.DMA,
        pltpu.SemaphoreType.DMA,
        pltpu.SemaphoreType.DMA,
        pltpu.SemaphoreType.DMA,
        pltpu.SemaphoreType.DMA,
    ],
)

_sc2 = pl.kernel(
    _sc2_body,
    compiler_params=_SC_PARAMS,
    out_type=jax.ShapeDtypeStruct((NCORE, N, D_EMB), jnp.float32),
    mesh=_SC_MESH,
    scratch_types=[
        pltpu.VMEM((NCH, CH), jnp.int32),
        pltpu.VMEM((NCH, CH), jnp.int32),
        pltpu.VMEM((EPT,), jnp.float32),
        pltpu.VMEM((CH, D_EMB), jnp.float32),
        pltpu.VMEM((CH, D_EMB), jnp.float32),
        pltpu.VMEM((CH, D_EMB), jnp.float32),
        pltpu.VMEM((CH, D_EMB), jnp.float32),
        pltpu.VMEM((CH, D_EMB), jnp.float32),
        pltpu.VMEM((NCH, D_EMB), jnp.float32),
        pltpu.VMEM_SHARED((N, D_EMB), jnp.float32),
        pltpu.SemaphoreType.DMA,
        pltpu.SemaphoreType.DMA,
        pltpu.SemaphoreType.DMA,
        pltpu.SemaphoreType.DMA,
        pltpu.SemaphoreType.DMA,
        pltpu.SemaphoreType.DMA,
        pltpu.SemaphoreType.DMA,
        pltpu.SemaphoreType.DMA,
        pltpu.SemaphoreType.DMA,
        pltpu.SemaphoreType.DMA,
    ],
)


def kernel(x, W1, S1, att_src1, att_dst1, W2, graph_edges):
    src3 = graph_edges[0].reshape(E // CH, CH)
    dst3 = graph_edges[1].reshape(E // CH, CH)

    h_src, a_src, a_dst = _enc_stage(x, W1, S1, att_src1, att_dst1)
    w_all, s, acc1 = _sc1(src3, dst3, a_src.reshape(N), a_dst.reshape(N),
                          h_src)
    s0 = s[0].reshape(N, 1)
    s1 = s[1].reshape(N, 1)
    emb = _mid_stage(acc1[0], acc1[1], s0, s1, W2)
    acc2 = _sc2(src3, dst3, w_all, emb)
    rec = _dec_stage(acc2[0], acc2[1], s0, s1, W2, W1)
    return emb, rec


# speedup vs baseline: 48.2543x; 1.0268x over previous
import jax
import jax.numpy as jnp
from jax import lax
from jax.experimental import pallas as pl
from jax.experimental.pallas import tpu as pltpu
from jax.experimental.pallas import tpu_sc as plsc

N = 10000
E = 320000
D_IN, D_LAT, D_EMB = 128, 64, 32

NCORE, NSUB = 2, 16
CH = 80
NCH = 125
EPT = CH * NCH
RSTRIPE = N // NSUB
TBLK = 1000


def _enc_body(x_ref, w1_ref, s1_ref, avs_ref, avd_ref, hs_ref, as_ref, ad_ref):
    xb = x_ref[...]
    hs = jnp.dot(xb, w1_ref[...], preferred_element_type=jnp.float32)
    hd = jnp.dot(xb, s1_ref[...], preferred_element_type=jnp.float32)
    hs_ref[...] = hs
    as_ref[...] = jnp.sum(hs * avs_ref[...][None, :], axis=1)[:, None]
    ad_ref[...] = jnp.sum(hd * avd_ref[...][None, :], axis=1)[:, None]


def _enc_stage(x, W1, S1, avs, avd):
    grid = (N // TBLK,)
    return pl.pallas_call(
        _enc_body,
        grid=grid,
        in_specs=[
            pl.BlockSpec((TBLK, D_IN), lambda i: (i, 0)),
            pl.BlockSpec((D_IN, D_LAT), lambda i: (0, 0)),
            pl.BlockSpec((D_IN, D_LAT), lambda i: (0, 0)),
            pl.BlockSpec((D_LAT,), lambda i: (0,)),
            pl.BlockSpec((D_LAT,), lambda i: (0,)),
        ],
        out_specs=[
            pl.BlockSpec((TBLK, D_LAT), lambda i: (i, 0)),
            pl.BlockSpec((TBLK, 1), lambda i: (i, 0)),
            pl.BlockSpec((TBLK, 1), lambda i: (i, 0)),
        ],
        out_shape=[
            jax.ShapeDtypeStruct((N, D_LAT), jnp.float32),
            jax.ShapeDtypeStruct((N, 1), jnp.float32),
            jax.ShapeDtypeStruct((N, 1), jnp.float32),
        ],
    )(x, W1, S1, avs, avd)


def _mid_body(p0_ref, p1_ref, s0_ref, s1_ref, w2_ref, emb_ref):
    t = (p0_ref[...] + p1_ref[...]) / (s0_ref[...] + s1_ref[...] + 1e-16)
    h1 = jnp.where(t > 0, t, jnp.exp(t) - 1.0)
    emb_ref[...] = jnp.dot(h1, w2_ref[...],
                           preferred_element_type=jnp.float32)


def _mid_stage(p0, p1, s0, s1, W2):
    grid = (N // TBLK,)
    return pl.pallas_call(
        _mid_body,
        grid=grid,
        in_specs=[
            pl.BlockSpec((TBLK, D_LAT), lambda i: (i, 0)),
            pl.BlockSpec((TBLK, D_LAT), lambda i: (i, 0)),
            pl.BlockSpec((TBLK, 1), lambda i: (i, 0)),
            pl.BlockSpec((TBLK, 1), lambda i: (i, 0)),
            pl.BlockSpec((D_LAT, D_EMB), lambda i: (0, 0)),
        ],
        out_specs=pl.BlockSpec((TBLK, D_EMB), lambda i: (i, 0)),
        out_shape=jax.ShapeDtypeStruct((N, D_EMB), jnp.float32),
    )(p0, p1, s0, s1, W2)


def _dec_body(p0_ref, p1_ref, s0_ref, s1_ref, w2_ref, w1_ref, rec_ref):
    u = (p0_ref[...] + p1_ref[...]) / (s0_ref[...] + s1_ref[...] + 1e-16)
    t = lax.dot_general(u, w2_ref[...], (((1,), (1,)), ((), ())),
                        preferred_element_type=jnp.float32)
    d1 = jnp.where(t > 0, t, jnp.exp(t) - 1.0)
    rec_ref[...] = lax.dot_general(d1, w1_ref[...], (((1,), (1,)), ((), ())),
                                   preferred_element_type=jnp.float32)


def _dec_stage(p0, p1, s0, s1, W2, W1):
    grid = (N // TBLK,)
    return pl.pallas_call(
        _dec_body,
        grid=grid,
        in_specs=[
            pl.BlockSpec((TBLK, D_EMB), lambda i: (i, 0)),
            pl.BlockSpec((TBLK, D_EMB), lambda i: (i, 0)),
            pl.BlockSpec((TBLK, 1), lambda i: (i, 0)),
            pl.BlockSpec((TBLK, 1), lambda i: (i, 0)),
            pl.BlockSpec((D_LAT, D_EMB), lambda i: (0, 0)),
            pl.BlockSpec((D_IN, D_LAT), lambda i: (0, 0)),
        ],
        out_specs=pl.BlockSpec((TBLK, D_IN), lambda i: (i, 0)),
        out_shape=jax.ShapeDtypeStruct((N, D_IN), jnp.float32),
    )(p0, p1, s0, s1, W2, W1)


def _zero_vmem(ref, nrow):
    z = jnp.zeros((16,), jnp.float32)

    def body(r, _):
        for q in range(ref.shape[1] // 16):
            ref[r, pl.ds(q * 16, 16)] = z
        return 0

    lax.fori_loop(0, nrow, body, 0)


NB = 5


def _p2_ring(src2_v, dst2_v, w1_v, rows, gsems, ssems, feat_hbm, acc_sh, d,
             pre=None):
    def g_issue(ch, b):
        pltpu.async_copy(feat_hbm.at[src2_v.at[ch]], rows[b], gsems[b])

    def g_wait(b):
        pltpu.make_async_copy(feat_hbm.at[src2_v.at[0]], rows[b],
                              gsems[b]).wait()

    def s_issue(ch, b):
        pltpu.async_copy(rows[b], acc_sh.at[dst2_v.at[ch]], ssems[b],
                         add=True)

    def s_wait(b):
        pltpu.make_async_copy(rows[b], acc_sh.at[dst2_v.at[0]],
                              ssems[b]).wait()

    def scale(ch, b):
        for g in range(CH // 16):
            wvec = w1_v[pl.ds(ch * CH + g * 16, 16)]
            for k in range(16):
                wv = jnp.full((16,), wvec[k])
                e = g * 16 + k
                for q in range(d // 16):
                    rows[b][e, pl.ds(q * 16, 16)] = (
                        rows[b][e, pl.ds(q * 16, 16)] * wv)

    g_issue(0, 0)
    g_issue(1, 1)
    g_issue(2, 2)

    def group(g, _):
        for b in range(NB):
            ch = g * NB + b

            @pl.when(ch + 3 < NCH)
            def _():
                nb = (b + 3) % NB

                @pl.when(ch >= 2)
                def _():
                    s_wait(nb)

                g_issue(ch + 3, nb)

            if pre is not None:
                pre(ch)
            g_wait(b)
            scale(ch, b)
            s_issue(ch, b)
        return 0

    lax.fori_loop(0, NCH // NB, group, 0)
    for b in range(NB):
        s_wait(b)


def _sc1_body(src3_hbm, dst3_hbm, asrc_hbm, adst_hbm, hsrc_hbm,
              w_hbm, s_hbm, acc_hbm,
              asrc_v, adst_v, src2_v, dst2_v, w1_v,
              r0_v, r1_v, r2_v, r3_v, r4_v, zb_v,
              s_sh, acc_sh,
              g0, g1, g2, g3, g4, s0, s1, s2, s3, s4, sem_s):
    c = lax.axis_index("c")
    t = lax.axis_index("s")

    pltpu.sync_copy(asrc_hbm, asrc_v)
    pltpu.sync_copy(adst_hbm, adst_v)

    _zero_vmem(zb_v, NCH)
    for k5 in range(RSTRIPE // NCH):
        pltpu.sync_copy(zb_v, acc_sh.at[pl.ds(t * RSTRIPE + k5 * NCH, NCH)])

    def zs(j, _):
        w1_v[pl.ds(j * 16, 16)] = jnp.zeros((16,), jnp.float32)
        return 0

    lax.fori_loop(0, 40, zs, 0)

    @pl.when(t < NSUB - 1)
    def _():
        pltpu.sync_copy(w1_v.at[pl.ds(0, 632)], s_sh.at[pl.ds(t * 632, 632)])

    @pl.when(t == NSUB - 1)
    def _():
        pltpu.sync_copy(w1_v.at[pl.ds(0, 520)], s_sh.at[pl.ds(15 * 632, 520)])

    plsc.subcore_barrier()

    wid = t * 2 + c
    pltpu.sync_copy(src3_hbm.at[pl.ds(wid * NCH, NCH)], src2_v)
    pltpu.sync_copy(dst3_hbm.at[pl.ds(wid * NCH, NCH)], dst2_v)

    def w_chunk(ch):
        for j in range(CH // 16):
            s16 = src2_v[ch, pl.ds(j * 16, 16)]
            d16 = dst2_v[ch, pl.ds(j * 16, 16)]
            a_s = plsc.load_gather(asrc_v, [s16])
            a_d = plsc.load_gather(adst_v, [d16])
            z = a_s + a_d
            sg = 1.0 / (1.0 + jnp.exp(-z))
            w1_v[pl.ds(ch * CH + j * 16, 16)] = jnp.exp(sg)
        pltpu.async_copy(w1_v.at[pl.ds(ch * CH, CH)],
                         s_sh.at[dst2_v.at[ch]], sem_s, add=True)

    _p2_ring(src2_v, dst2_v, w1_v, [r0_v, r1_v, r2_v, r3_v, r4_v],
             [g0, g1, g2, g3, g4], [s0, s1, s2, s3, s4],
             hsrc_hbm, acc_sh, D_LAT, pre=w_chunk)

    pltpu.sync_copy(w1_v, w_hbm.at[pl.ds(wid * EPT, EPT)])

    def p1_drain(i, _):
        pltpu.make_async_copy(w1_v.at[pl.ds(0, CH)],
                              s_sh.at[dst2_v.at[0]], sem_s).wait()
        return 0

    lax.fori_loop(0, NCH, p1_drain, 0)

    plsc.subcore_barrier()

    r0 = t * RSTRIPE
    pltpu.sync_copy(acc_sh.at[pl.ds(r0, RSTRIPE)],
                    acc_hbm.at[c, pl.ds(r0, RSTRIPE)])

    @pl.when(t < NSUB - 1)
    def _():
        pltpu.sync_copy(s_sh.at[pl.ds(t * 632, 632)],
                        s_hbm.at[c, pl.ds(t * 632, 632)])

    @pl.when(t == NSUB - 1)
    def _():
        pltpu.sync_copy(s_sh.at[pl.ds(15 * 632, 520)],
                        s_hbm.at[c, pl.ds(15 * 632, 520)])


def _sc2_body(src3_hbm, dst3_hbm, w_all_hbm, feat_hbm,
              acc_hbm,
              src2_v, dst2_v, w1_v,
              r0_v, r1_v, r2_v, r3_v, r4_v, zb_v, acc_sh,
              g0, g1, g2, g3, g4, s0, s1, s2, s3, s4):
    c = lax.axis_index("c")
    t = lax.axis_index("s")

    _zero_vmem(zb_v, NCH)
    for k5 in range(RSTRIPE // NCH):
        pltpu.sync_copy(zb_v, acc_sh.at[pl.ds(t * RSTRIPE + k5 * NCH, NCH)])
    plsc.subcore_barrier()

    wid = t * 2 + c
    pltpu.sync_copy(src3_hbm.at[pl.ds(wid * NCH, NCH)], src2_v)
    pltpu.sync_copy(dst3_hbm.at[pl.ds(wid * NCH, NCH)], dst2_v)
    pltpu.sync_copy(w_all_hbm.at[pl.ds(wid * EPT, EPT)], w1_v)

    _p2_ring(src2_v, dst2_v, w1_v, [r0_v, r1_v, r2_v, r3_v, r4_v],
             [g0, g1, g2, g3, g4], [s0, s1, s2, s3, s4],
             feat_hbm, acc_sh, D_EMB)

    plsc.subcore_barrier()
    r0 = t * RSTRIPE
    pltpu.sync_copy(acc_sh.at[pl.ds(r0, RSTRIPE)],
                    acc_hbm.at[c, pl.ds(r0, RSTRIPE)])


_SC_MESH = plsc.VectorSubcoreMesh(core_axis_name="c", subcore_axis_name="s")
_SC_PARAMS = pltpu.CompilerParams(needs_layout_passes=False,
                                  use_tc_tiling_on_sc=False)

_sc1 = pl.kernel(
    _sc1_body,
    compiler_params=_SC_PARAMS,
    out_type=[
        jax.ShapeDtypeStruct((E,), jnp.float32),
        jax.ShapeDtypeStruct((NCORE, N), jnp.float32),
        jax.ShapeDtypeStruct((NCORE, N, D_LAT), jnp.float32),
    ],
    mesh=_SC_MESH,
    scratch_types=[
        pltpu.VMEM((N,), jnp.float32),
        pltpu.VMEM((N,), jnp.float32),
        pltpu.VMEM((NCH, CH), jnp.int32),
        pltpu.VMEM((NCH, CH), jnp.int32),
        pltpu.VMEM((EPT,), jnp.float32),
        pltpu.VMEM((CH, D_LAT), jnp.float32),
        pltpu.VMEM((CH, D_LAT), jnp.float32),
        pltpu.VMEM((CH, D_LAT), jnp.float32),
        pltpu.VMEM((CH, D_LAT), jnp.float32),
        pltpu.VMEM((CH, D_LAT), jnp.float32),
        pltpu.VMEM((NCH, D_LAT), jnp.float32),
        pltpu.VMEM_SHARED((N,), jnp.float32),
        pltpu.VMEM_SHARED((N, D_LAT), jnp.float32),
        pltpu.SemaphoreType.DMA,
        pltpu.SemaphoreType.DMA,
        pltpu.SemaphoreType.DMA,
        pltpu.SemaphoreType.DMA,
        pltpu.SemaphoreType.DMA,
        pltpu.SemaphoreType.DMA,
        pltpu.SemaphoreType.DMA,
        pltpu.SemaphoreType.DMA,
        pltpu.SemaphoreType.DMA,
        pltpu.SemaphoreType.DMA,
        pltpu.SemaphoreType.DMA,
    ],
)

_sc2 = pl.kernel(
    _sc2_body,
    compiler_params=_SC_PARAMS,
    out_type=jax.ShapeDtypeStruct((NCORE, N, D_EMB), jnp.float32),
    mesh=_SC_MESH,
    scratch_types=[
        pltpu.VMEM((NCH, CH), jnp.int32),
        pltpu.VMEM((NCH, CH), jnp.int32),
        pltpu.VMEM((EPT,), jnp.float32),
        pltpu.VMEM((CH, D_EMB), jnp.float32),
        pltpu.VMEM((CH, D_EMB), jnp.float32),
        pltpu.VMEM((CH, D_EMB), jnp.float32),
        pltpu.VMEM((CH, D_EMB), jnp.float32),
        pltpu.VMEM((CH, D_EMB), jnp.float32),
        pltpu.VMEM((NCH, D_EMB), jnp.float32),
        pltpu.VMEM_SHARED((N, D_EMB), jnp.float32),
        pltpu.SemaphoreType.DMA,
        pltpu.SemaphoreType.DMA,
        pltpu.SemaphoreType.DMA,
        pltpu.SemaphoreType.DMA,
        pltpu.SemaphoreType.DMA,
        pltpu.SemaphoreType.DMA,
        pltpu.SemaphoreType.DMA,
        pltpu.SemaphoreType.DMA,
        pltpu.SemaphoreType.DMA,
        pltpu.SemaphoreType.DMA,
    ],
)


def kernel(x, W1, S1, att_src1, att_dst1, W2, graph_edges):
    src3 = graph_edges[0].reshape(E // CH, CH)
    dst3 = graph_edges[1].reshape(E // CH, CH)

    h_src, a_src, a_dst = _enc_stage(x, W1, S1, att_src1, att_dst1)
    w_all, s, acc1 = _sc1(src3, dst3, a_src.reshape(N), a_dst.reshape(N),
                          h_src)
    s0 = s[0].reshape(N, 1)
    s1 = s[1].reshape(N, 1)
    emb = _mid_stage(acc1[0], acc1[1], s0, s1, W2)
    acc2 = _sc2(src3, dst3, w_all, emb)
    rec = _dec_stage(acc2[0], acc2[1], s0, s1, W2, W1)
    return emb, rec


# speedup vs baseline: 49.9188x; 1.0345x over previous
import jax
import jax.numpy as jnp
from jax import lax
from jax.experimental import pallas as pl
from jax.experimental.pallas import tpu as pltpu
from jax.experimental.pallas import tpu_sc as plsc

N = 10000
E = 320000
D_IN, D_LAT, D_EMB = 128, 64, 32

NCORE, NSUB = 2, 16
CH = 80
NCH = 125
EPT = CH * NCH
RSTRIPE = N // NSUB
TBLK = 1000


def _enc_body(x_ref, w1_ref, s1_ref, avs_ref, avd_ref, hs_ref, as_ref, ad_ref):
    xb = x_ref[...]
    hs = jnp.dot(xb, w1_ref[...], preferred_element_type=jnp.float32)
    hd = jnp.dot(xb, s1_ref[...], preferred_element_type=jnp.float32)
    hs_ref[...] = hs
    as_ref[...] = jnp.sum(hs * avs_ref[...][None, :], axis=1)[:, None]
    ad_ref[...] = jnp.sum(hd * avd_ref[...][None, :], axis=1)[:, None]


def _enc_stage(x, W1, S1, avs, avd):
    grid = (N // TBLK,)
    return pl.pallas_call(
        _enc_body,
        grid=grid,
        in_specs=[
            pl.BlockSpec((TBLK, D_IN), lambda i: (i, 0)),
            pl.BlockSpec((D_IN, D_LAT), lambda i: (0, 0)),
            pl.BlockSpec((D_IN, D_LAT), lambda i: (0, 0)),
            pl.BlockSpec((D_LAT,), lambda i: (0,)),
            pl.BlockSpec((D_LAT,), lambda i: (0,)),
        ],
        out_specs=[
            pl.BlockSpec((TBLK, D_LAT), lambda i: (i, 0)),
            pl.BlockSpec((TBLK, 1), lambda i: (i, 0)),
            pl.BlockSpec((TBLK, 1), lambda i: (i, 0)),
        ],
        out_shape=[
            jax.ShapeDtypeStruct((N, D_LAT), jnp.float32),
            jax.ShapeDtypeStruct((N, 1), jnp.float32),
            jax.ShapeDtypeStruct((N, 1), jnp.float32),
        ],
    )(x, W1, S1, avs, avd)


def _mid_body(p0_ref, p1_ref, s0_ref, s1_ref, w2_ref, emb_ref):
    t = (p0_ref[...] + p1_ref[...]) / (s0_ref[...] + s1_ref[...] + 1e-16)
    h1 = jnp.where(t > 0, t, jnp.exp(t) - 1.0)
    emb_ref[...] = jnp.dot(h1, w2_ref[...],
                           preferred_element_type=jnp.float32)


def _mid_stage(p0, p1, s0, s1, W2):
    grid = (N // TBLK,)
    return pl.pallas_call(
        _mid_body,
        grid=grid,
        in_specs=[
            pl.BlockSpec((TBLK, D_LAT), lambda i: (i, 0)),
            pl.BlockSpec((TBLK, D_LAT), lambda i: (i, 0)),
            pl.BlockSpec((TBLK, 1), lambda i: (i, 0)),
            pl.BlockSpec((TBLK, 1), lambda i: (i, 0)),
            pl.BlockSpec((D_LAT, D_EMB), lambda i: (0, 0)),
        ],
        out_specs=pl.BlockSpec((TBLK, D_EMB), lambda i: (i, 0)),
        out_shape=jax.ShapeDtypeStruct((N, D_EMB), jnp.float32),
    )(p0, p1, s0, s1, W2)


def _dec_body(p0_ref, p1_ref, s0_ref, s1_ref, w2_ref, w1_ref, rec_ref):
    u = (p0_ref[...] + p1_ref[...]) / (s0_ref[...] + s1_ref[...] + 1e-16)
    t = lax.dot_general(u, w2_ref[...], (((1,), (1,)), ((), ())),
                        preferred_element_type=jnp.float32)
    d1 = jnp.where(t > 0, t, jnp.exp(t) - 1.0)
    rec_ref[...] = lax.dot_general(d1, w1_ref[...], (((1,), (1,)), ((), ())),
                                   preferred_element_type=jnp.float32)


def _dec_stage(p0, p1, s0, s1, W2, W1):
    grid = (N // TBLK,)
    return pl.pallas_call(
        _dec_body,
        grid=grid,
        in_specs=[
            pl.BlockSpec((TBLK, D_EMB), lambda i: (i, 0)),
            pl.BlockSpec((TBLK, D_EMB), lambda i: (i, 0)),
            pl.BlockSpec((TBLK, 1), lambda i: (i, 0)),
            pl.BlockSpec((TBLK, 1), lambda i: (i, 0)),
            pl.BlockSpec((D_LAT, D_EMB), lambda i: (0, 0)),
            pl.BlockSpec((D_IN, D_LAT), lambda i: (0, 0)),
        ],
        out_specs=pl.BlockSpec((TBLK, D_IN), lambda i: (i, 0)),
        out_shape=jax.ShapeDtypeStruct((N, D_IN), jnp.float32),
    )(p0, p1, s0, s1, W2, W1)


def _zero_vmem(ref, nrow):
    z = jnp.zeros((16,), jnp.float32)

    def body(r, _):
        for q in range(ref.shape[1] // 16):
            ref[r, pl.ds(q * 16, 16)] = z
        return 0

    lax.fori_loop(0, nrow, body, 0)


NB = 5


def _p2_ring(src2_v, dst2_v, w1_v, rows, gsems, ssems, feat_hbm, acc_sh, d,
             pre=None):
    def g_issue(ch, b):
        pltpu.async_copy(feat_hbm.at[src2_v.at[ch]], rows[b], gsems[b])

    def g_wait(b):
        pltpu.make_async_copy(feat_hbm.at[src2_v.at[0]], rows[b],
                              gsems[b]).wait()

    def s_issue(ch, b):
        pltpu.async_copy(rows[b], acc_sh.at[dst2_v.at[ch]], ssems[b],
                         add=True)

    def s_wait(b):
        pltpu.make_async_copy(rows[b], acc_sh.at[dst2_v.at[0]],
                              ssems[b]).wait()

    def scale(ch, b):
        for g in range(CH // 16):
            wvec = w1_v[pl.ds(ch * CH + g * 16, 16)]
            for k in range(16):
                wv = jnp.full((16,), wvec[k])
                e = g * 16 + k
                for q in range(d // 16):
                    rows[b][e, pl.ds(q * 16, 16)] = (
                        rows[b][e, pl.ds(q * 16, 16)] * wv)

    g_issue(0, 0)
    g_issue(1, 1)
    g_issue(2, 2)

    def group(g, _):
        for b in range(NB):
            ch = g * NB + b

            @pl.when(ch + 3 < NCH)
            def _():
                nb = (b + 3) % NB

                @pl.when(ch >= 2)
                def _():
                    s_wait(nb)

                g_issue(ch + 3, nb)

            if pre is not None:
                pre(ch)
            g_wait(b)
            scale(ch, b)
            s_issue(ch, b)
        return 0

    lax.fori_loop(0, NCH // NB, group, 0)
    for b in range(NB):
        s_wait(b)


def _sc1_body(src3_hbm, dst3_hbm, asrc_hbm, adst_hbm, hsrc_hbm,
              w_hbm, s_hbm, acc_hbm,
              asrc_v, adst_v, src2_v, dst2_v, w1_v,
              r0_v, r1_v, r2_v, r3_v, r4_v, zb_v,
              s_sh, acc_sh,
              g0, g1, g2, g3, g4, s0, s1, s2, s3, s4, sem_s):
    c = lax.axis_index("c")
    t = lax.axis_index("s")

    wid = t * 2 + c
    pltpu.async_copy(asrc_hbm, asrc_v, g0)
    pltpu.async_copy(adst_hbm, adst_v, g1)
    pltpu.async_copy(src3_hbm.at[pl.ds(wid * NCH, NCH)], src2_v, g2)
    pltpu.async_copy(dst3_hbm.at[pl.ds(wid * NCH, NCH)], dst2_v, g3)

    _zero_vmem(zb_v, NCH)
    for k5 in range(RSTRIPE // NCH):
        pltpu.async_copy(zb_v, acc_sh.at[pl.ds(t * RSTRIPE + k5 * NCH, NCH)],
                         g4)

    def zs(j, _):
        w1_v[pl.ds(j * 16, 16)] = jnp.zeros((16,), jnp.float32)
        return 0

    lax.fori_loop(0, 40, zs, 0)

    @pl.when(t < NSUB - 1)
    def _():
        pltpu.sync_copy(w1_v.at[pl.ds(0, 632)], s_sh.at[pl.ds(t * 632, 632)])

    @pl.when(t == NSUB - 1)
    def _():
        pltpu.sync_copy(w1_v.at[pl.ds(0, 520)], s_sh.at[pl.ds(15 * 632, 520)])

    pltpu.make_async_copy(asrc_hbm, asrc_v, g0).wait()
    pltpu.make_async_copy(adst_hbm, adst_v, g1).wait()
    pltpu.make_async_copy(src3_hbm.at[pl.ds(0, NCH)], src2_v, g2).wait()
    pltpu.make_async_copy(dst3_hbm.at[pl.ds(0, NCH)], dst2_v, g3).wait()
    for k5 in range(RSTRIPE // NCH):
        pltpu.make_async_copy(zb_v, acc_sh.at[pl.ds(0, NCH)], g4).wait()

    plsc.subcore_barrier()

    def w_chunk(ch):
        for j in range(CH // 16):
            s16 = src2_v[ch, pl.ds(j * 16, 16)]
            d16 = dst2_v[ch, pl.ds(j * 16, 16)]
            a_s = plsc.load_gather(asrc_v, [s16])
            a_d = plsc.load_gather(adst_v, [d16])
            z = a_s + a_d
            sg = 1.0 / (1.0 + jnp.exp(-z))
            w1_v[pl.ds(ch * CH + j * 16, 16)] = jnp.exp(sg)
        pltpu.async_copy(w1_v.at[pl.ds(ch * CH, CH)],
                         s_sh.at[dst2_v.at[ch]], sem_s, add=True)

    _p2_ring(src2_v, dst2_v, w1_v, [r0_v, r1_v, r2_v, r3_v, r4_v],
             [g0, g1, g2, g3, g4], [s0, s1, s2, s3, s4],
             hsrc_hbm, acc_sh, D_LAT, pre=w_chunk)

    pltpu.sync_copy(w1_v, w_hbm.at[pl.ds(wid * EPT, EPT)])

    def p1_drain(i, _):
        pltpu.make_async_copy(w1_v.at[pl.ds(0, CH)],
                              s_sh.at[dst2_v.at[0]], sem_s).wait()
        return 0

    lax.fori_loop(0, NCH, p1_drain, 0)

    plsc.subcore_barrier()

    r0 = t * RSTRIPE
    pltpu.sync_copy(acc_sh.at[pl.ds(r0, RSTRIPE)],
                    acc_hbm.at[c, pl.ds(r0, RSTRIPE)])

    @pl.when(t < NSUB - 1)
    def _():
        pltpu.sync_copy(s_sh.at[pl.ds(t * 632, 632)],
                        s_hbm.at[c, pl.ds(t * 632, 632)])

    @pl.when(t == NSUB - 1)
    def _():
        pltpu.sync_copy(s_sh.at[pl.ds(15 * 632, 520)],
                        s_hbm.at[c, pl.ds(15 * 632, 520)])


def _sc2_body(src3_hbm, dst3_hbm, w_all_hbm, feat_hbm,
              acc_hbm,
              src2_v, dst2_v, w1_v,
              r0_v, r1_v, r2_v, r3_v, r4_v, zb_v, acc_sh,
              g0, g1, g2, g3, g4, s0, s1, s2, s3, s4):
    c = lax.axis_index("c")
    t = lax.axis_index("s")

    wid = t * 2 + c
    pltpu.async_copy(src3_hbm.at[pl.ds(wid * NCH, NCH)], src2_v, g0)
    pltpu.async_copy(dst3_hbm.at[pl.ds(wid * NCH, NCH)], dst2_v, g1)
    pltpu.async_copy(w_all_hbm.at[pl.ds(wid * EPT, EPT)], w1_v, g2)

    _zero_vmem(zb_v, NCH)
    for k5 in range(RSTRIPE // NCH):
        pltpu.async_copy(zb_v, acc_sh.at[pl.ds(t * RSTRIPE + k5 * NCH, NCH)],
                         g3)

    pltpu.make_async_copy(src3_hbm.at[pl.ds(0, NCH)], src2_v, g0).wait()
    pltpu.make_async_copy(dst3_hbm.at[pl.ds(0, NCH)], dst2_v, g1).wait()
    pltpu.make_async_copy(w_all_hbm.at[pl.ds(0, EPT)], w1_v, g2).wait()
    for k5 in range(RSTRIPE // NCH):
        pltpu.make_async_copy(zb_v, acc_sh.at[pl.ds(0, NCH)], g3).wait()

    plsc.subcore_barrier()

    _p2_ring(src2_v, dst2_v, w1_v, [r0_v, r1_v, r2_v, r3_v, r4_v],
             [g0, g1, g2, g3, g4], [s0, s1, s2, s3, s4],
             feat_hbm, acc_sh, D_EMB)

    plsc.subcore_barrier()
    r0 = t * RSTRIPE
    pltpu.sync_copy(acc_sh.at[pl.ds(r0, RSTRIPE)],
                    acc_hbm.at[c, pl.ds(r0, RSTRIPE)])


_SC_MESH = plsc.VectorSubcoreMesh(core_axis_name="c", subcore_axis_name="s")
_SC_PARAMS = pltpu.CompilerParams(needs_layout_passes=False,
                                  use_tc_tiling_on_sc=False)

_sc1 = pl.kernel(
    _sc1_body,
    compiler_params=_SC_PARAMS,
    out_type=[
        jax.ShapeDtypeStruct((E,), jnp.float32),
        jax.ShapeDtypeStruct((NCORE, N), jnp.float32),
        jax.ShapeDtypeStruct((NCORE, N, D_LAT), jnp.float32),
    ],
    mesh=_SC_MESH,
    scratch_types=[
        pltpu.VMEM((N,), jnp.float32),
        pltpu.VMEM((N,), jnp.float32),
        pltpu.VMEM((NCH, CH), jnp.int32),
        pltpu.VMEM((NCH, CH), jnp.int32),
        pltpu.VMEM((EPT,), jnp.float32),
        pltpu.VMEM((CH, D_LAT), jnp.float32),
        pltpu.VMEM((CH, D_LAT), jnp.float32),
        pltpu.VMEM((CH, D_LAT), jnp.float32),
        pltpu.VMEM((CH, D_LAT), jnp.float32),
        pltpu.VMEM((CH, D_LAT), jnp.float32),
        pltpu.VMEM((NCH, D_LAT), jnp.float32),
        pltpu.VMEM_SHARED((N,), jnp.float32),
        pltpu.VMEM_SHARED((N, D_LAT), jnp.float32),
        pltpu.SemaphoreType.DMA,
        pltpu.SemaphoreType.DMA,
        pltpu.SemaphoreType.DMA,
        pltpu.SemaphoreType.DMA,
        pltpu.SemaphoreType.DMA,
        pltpu.SemaphoreType.DMA,
        pltpu.SemaphoreType.DMA,
        pltpu.SemaphoreType.DMA,
        pltpu.SemaphoreType.DMA,
        pltpu.SemaphoreType.DMA,
        pltpu.SemaphoreType.DMA,
    ],
)

_sc2 = pl.kernel(
    _sc2_body,
    compiler_params=_SC_PARAMS,
    out_type=jax.ShapeDtypeStruct((NCORE, N, D_EMB), jnp.float32),
    mesh=_SC_MESH,
    scratch_types=[
        pltpu.VMEM((NCH, CH), jnp.int32),
        pltpu.VMEM((NCH, CH), jnp.int32),
        pltpu.VMEM((EPT,), jnp.float32),
        pltpu.VMEM((CH, D_EMB), jnp.float32),
        pltpu.VMEM((CH, D_EMB), jnp.float32),
        pltpu.VMEM((CH, D_EMB), jnp.float32),
        pltpu.VMEM((CH, D_EMB), jnp.float32),
        pltpu.VMEM((CH, D_EMB), jnp.float32),
        pltpu.VMEM((NCH, D_EMB), jnp.float32),
        pltpu.VMEM_SHARED((N, D_EMB), jnp.float32),
        pltpu.SemaphoreType.DMA,
        pltpu.SemaphoreType.DMA,
        pltpu.SemaphoreType.DMA,
        pltpu.SemaphoreType.DMA,
        pltpu.SemaphoreType.DMA,
        pltpu.SemaphoreType.DMA,
        pltpu.SemaphoreType.DMA,
        pltpu.SemaphoreType.DMA,
        pltpu.SemaphoreType.DMA,
        pltpu.SemaphoreType.DMA,
    ],
)


def kernel(x, W1, S1, att_src1, att_dst1, W2, graph_edges):
    src3 = graph_edges[0].reshape(E // CH, CH)
    dst3 = graph_edges[1].reshape(E // CH, CH)

    h_src, a_src, a_dst = _enc_stage(x, W1, S1, att_src1, att_dst1)
    w_all, s, acc1 = _sc1(src3, dst3, a_src.reshape(N), a_dst.reshape(N),
                          h_src)
    s0 = s[0].reshape(N, 1)
    s1 = s[1].reshape(N, 1)
    emb = _mid_stage(acc1[0], acc1[1], s0, s1, W2)
    acc2 = _sc2(src3, dst3, w_all, emb)
    rec = _dec_stage(acc2[0], acc2[1], s0, s1, W2, W1)
    return emb, rec
